# Initial kernel scaffold; baseline (speedup 1.0000x reference)
import jax
import jax.numpy as jnp
from jax.experimental import pallas as pl


def kernel(input, target):
    raise NotImplementedError("write your pallas kernel here")


# speedup vs baseline: 3.6951x; 3.6951x over previous
import functools

import jax
import jax.numpy as jnp
from jax import lax
from jax.experimental import pallas as pl
from jax.experimental.pallas import tpu as pltpu
from jax.experimental.pallas import tpu_sc as plsc

L16 = 16
NEG = -1.0e18
POS = 1.0e18


def _chamfer_sc(csort_pad, tflat, *, n, p, hw):
    n_tiles = 16
    tiles_per_row = n_tiles // n
    chunk = hw // tiles_per_row
    groups = chunk // L16
    cpad = csort_pad.shape[1]
    nseg = p + 1
    seg_pad = 304
    trash = 299
    rec = 640
    a_groups = cpad // L16
    c_groups = p // L16

    mesh = plsc.VectorSubcoreMesh(
        core_axis_name="c", subcore_axis_name="s", num_cores=1,
        num_subcores=n_tiles,
    )

    @functools.partial(
        pl.kernel,
        out_type=jax.ShapeDtypeStruct((L16,), jnp.float32),
        mesh=mesh,
        compiler_params=pltpu.CompilerParams(
            use_tc_tiling_on_sc=False, needs_layout_passes=False),
        scratch_types=dict(
            vals_v=pltpu.VMEM((chunk,), jnp.float32),
            cs_v=pltpu.VMEM((cpad,), jnp.float32),
            imax_v=pltpu.VMEM((seg_pad,), jnp.float32),
            imin_v=pltpu.VMEM((seg_pad,), jnp.float32),
            sbuf_v=pltpu.VMEM((32,), jnp.int32),
            stats_v=pltpu.VMEM((L16,), jnp.float32),
            row_v=pltpu.VMEM((tiles_per_row, rec), jnp.float32),
            cimax_v=pltpu.VMEM((cpad,), jnp.float32),
            cimin_v=pltpu.VMEM((cpad,), jnp.float32),
            pmax_v=pltpu.VMEM((cpad,), jnp.float32),
            smin_v=pltpu.VMEM((cpad,), jnp.float32),
            res_v=pltpu.VMEM((n, L16), jnp.float32),
            out_v=pltpu.VMEM((L16,), jnp.float32),
            shared=pltpu.VMEM_SHARED((n_tiles, rec), jnp.float32),
        ),
    )
    def k(cs_hbm, t_hbm, out_hbm, vals_v, cs_v, imax_v, imin_v, sbuf_v,
          stats_v, row_v, cimax_v, cimin_v, pmax_v, smin_v, res_v, out_v,
          shared):
        wid = lax.axis_index("s")
        row = wid // tiles_per_row
        slot = wid % tiles_per_row
        base = row * hw + slot * chunk

        pltpu.sync_copy(t_hbm.at[pl.ds(base, chunk)], vals_v)
        pltpu.sync_copy(cs_hbm.at[row], cs_v)

        fneg = jnp.full((L16,), NEG, jnp.float32)
        fpos = jnp.full((L16,), POS, jnp.float32)
        for g in range(seg_pad // L16):
            imax_v[pl.ds(g * L16, L16)] = fneg
            imin_v[pl.ds(g * L16, L16)] = fpos
        im1 = jnp.full((L16,), -1, jnp.int32)
        sbuf_v[pl.ds(0, L16)] = im1
        sbuf_v[pl.ds(L16, L16)] = im1

        def body(g, carry):
            b_acc, l_acc = carry
            v = vals_v[pl.ds(g * L16, L16)]
            valid = v >= 0.001
            l_acc = l_acc + jnp.where(valid, 1.0, 0.0).astype(jnp.float32)
            lo = jnp.zeros((L16,), jnp.int32)
            hi = jnp.full((L16,), p, jnp.int32)
            for _ in range(9):
                mid = (lo + hi) >> 1
                cm = plsc.load_gather(cs_v, [mid + 1])
                go = cm <= v
                lo = jnp.where(go, mid + 1, lo)
                hi = jnp.where(go, hi, mid)
            s = lo
            lower = plsc.load_gather(cs_v, [s])
            upper = plsc.load_gather(cs_v, [s + 1])
            d = jnp.minimum(v - lower, upper - v)
            b_acc = b_acc + jnp.where(valid, d * d, 0.0)
            key = jnp.where(valid, v, 2.0)
            sidx = jnp.where(valid, s, trash)
            ksort, isort = plsc.sort_key_val(key, sidx)
            sbuf_v[pl.ds(1, L16)] = isort
            sprev = sbuf_v[pl.ds(0, L16)]
            snext = sbuf_v[pl.ds(2, L16)]
            m_last = isort != snext
            m_first = isort != sprev
            old_mx = plsc.load_gather(imax_v, [isort])
            plsc.store_scatter(imax_v, [isort], jnp.maximum(old_mx, ksort),
                               mask=m_last)
            old_mn = plsc.load_gather(imin_v, [isort])
            plsc.store_scatter(imin_v, [isort], jnp.minimum(old_mn, ksort),
                               mask=m_first)
            return b_acc, l_acc

        zero16 = jnp.zeros((L16,), jnp.float32)
        b_acc, l_acc = lax.fori_loop(0, groups, body, (zero16, zero16))

        ii = lax.iota(jnp.int32, L16)
        b_sum = jnp.sum(b_acc)
        l_sum = jnp.sum(l_acc)
        stats_v[...] = jnp.where(ii == 0, b_sum,
                                 jnp.where(ii == 1, l_sum, 0.0)
                                 ).astype(jnp.float32)
        pltpu.sync_copy(imax_v, shared.at[wid, pl.ds(0, seg_pad)])
        pltpu.sync_copy(imin_v, shared.at[wid, pl.ds(seg_pad, seg_pad)])
        pltpu.sync_copy(stats_v, shared.at[wid, pl.ds(608, L16)])
        plsc.subcore_barrier()

        @pl.when(wid < n)
        def _():
            r = wid
            pltpu.sync_copy(shared.at[pl.ds(r * tiles_per_row, tiles_per_row)],
                            row_v)
            pltpu.sync_copy(cs_hbm.at[r], cs_v)
            for g in range(a_groups):
                mx = row_v[0, pl.ds(g * L16, L16)]
                mn = row_v[0, pl.ds(seg_pad + g * L16, L16)]
                for t in range(1, tiles_per_row):
                    mx = jnp.maximum(mx, row_v[t, pl.ds(g * L16, L16)])
                    mn = jnp.minimum(mn, row_v[t, pl.ds(seg_pad + g * L16, L16)])
                cimax_v[pl.ds(g * L16, L16)] = mx
                cimin_v[pl.ds(g * L16, L16)] = mn
            st = row_v[0, pl.ds(608, L16)]
            for t in range(1, tiles_per_row):
                st = st + row_v[t, pl.ds(608, L16)]
            b_r = st[0]
            l_r = st[1]
            carry = jnp.float32(NEG)
            for g in range(a_groups):
                x = cimax_v[pl.ds(g * L16, L16)]
                pmax_v[pl.ds(g * L16, L16)] = jnp.maximum(plsc.cummax(x), carry)
                carry = jnp.maximum(carry, jnp.max(x))
            carry = jnp.float32(POS)
            for g in reversed(range(a_groups)):
                x = cimin_v[pl.ds(g * L16, L16)]
                suff = -lax.rev(plsc.cummax(lax.rev(-x, (0,))), (0,))
                smin_v[pl.ds(g * L16, L16)] = jnp.minimum(suff, carry)
                carry = jnp.minimum(carry, jnp.min(x))
            sa = jnp.zeros((L16,), jnp.float32)
            sac = jnp.zeros((L16,), jnp.float32)
            for h in range(c_groups):
                cp = cs_v[pl.ds(1 + h * L16, L16)]
                nb = pmax_v[pl.ds(h * L16, L16)]
                na = smin_v[pl.ds(h * L16 + 1, L16)]
                db = cp - nb
                da = na - cp
                a_p = jnp.minimum(db * db, da * da)
                sa = sa + a_p
                sac = sac + jnp.minimum(a_p, cp * cp)
            sa_s = jnp.sum(sa)
            sac_s = jnp.sum(sac)
            c0 = cs_v[pl.ds(0, L16)][1]
            w = jnp.where(ii == 0, sa_s,
                          jnp.where(ii == 1, sac_s,
                                    jnp.where(ii == 2, b_r,
                                              jnp.where(ii == 3, l_r,
                                                        jnp.where(ii == 4, c0,
                                                                  0.0)))))
            stats_v[...] = w.astype(jnp.float32)
            pltpu.sync_copy(stats_v, shared.at[wid, pl.ds(624, L16)])

        plsc.subcore_barrier()

        @pl.when(wid == 0)
        def _():
            pltpu.sync_copy(shared.at[pl.ds(0, n), pl.ds(624, L16)], res_v)
            rv = [res_v[r, pl.ds(0, L16)] for r in range(n)]
            ls = [rv[r][3] for r in range(n)]
            t_max = ls[0]
            for r in range(1, n):
                t_max = jnp.maximum(t_max, ls[r])
            chx = jnp.float32(0.0)
            chy_num = jnp.float32(0.0)
            for r in range(n):
                sa_s = rv[r][0]
                sac_s = rv[r][1]
                b_r = rv[r][2]
                c0 = rv[r][4]
                chx = chx + jnp.where(t_max > ls[r], sac_s, sa_s) * (1.0 / p)
                chy_num = chy_num + b_r + (t_max - ls[r]) * c0 * c0
            chy_vec = (jnp.broadcast_to(chy_num, (L16,))
                       / jnp.broadcast_to(t_max, (L16,)))
            loss_vec = (0.1 / n) * (chx + chy_vec)
            out_v[...] = jnp.where(ii == 0, loss_vec, 0.0).astype(jnp.float32)
            pltpu.sync_copy(out_v, out_hbm)

    return k(csort_pad, tflat)


def kernel(input, target):
    bins = input
    centers = 0.5 * (bins[:, 1:] + bins[:, :-1])
    n, p = centers.shape
    hw = target.shape[1] * target.shape[2]
    csort = jnp.sort(centers, axis=1)
    csort_pad = jnp.concatenate(
        [jnp.full((n, 1), NEG, jnp.float32),
         csort,
         jnp.full((n, 15), POS, jnp.float32)], axis=1)
    tflat = target.reshape(n * hw)
    out = _chamfer_sc(csort_pad, tflat, n=n, p=p, hw=hw)
    return out[0]


# speedup vs baseline: 6.0683x; 1.6423x over previous
import functools

import jax
import jax.numpy as jnp
from jax import lax
from jax.experimental import pallas as pl
from jax.experimental.pallas import tpu as pltpu
from jax.experimental.pallas import tpu_sc as plsc

L16 = 16
NEG = -1.0e18
POS = 1.0e18
SEG_PAD = 304
REC = 640


def _pixel_phase_sc(csort_pad, tflat, *, n, p, hw, n_tiles):
    tiles_per_row = n_tiles // n
    chunk = hw // tiles_per_row
    groups = chunk // L16
    cpad = csort_pad.shape[1]
    trash = 299

    mesh = plsc.VectorSubcoreMesh(
        core_axis_name="c", subcore_axis_name="s", num_cores=2,
        num_subcores=16,
    )

    @functools.partial(
        pl.kernel,
        out_type=jax.ShapeDtypeStruct((n_tiles, REC), jnp.float32),
        mesh=mesh,
        compiler_params=pltpu.CompilerParams(
            use_tc_tiling_on_sc=False, needs_layout_passes=False),
        scratch_types=dict(
            vals_v=pltpu.VMEM((chunk,), jnp.float32),
            cs_v=pltpu.VMEM((cpad,), jnp.float32),
            imax_v=pltpu.VMEM((SEG_PAD,), jnp.float32),
            imin_v=pltpu.VMEM((SEG_PAD,), jnp.float32),
            sbuf_v=pltpu.VMEM((32,), jnp.int32),
            stats_v=pltpu.VMEM((L16,), jnp.float32),
        ),
    )
    def k(cs_hbm, t_hbm, recs_hbm, vals_v, cs_v, imax_v, imin_v, sbuf_v,
          stats_v):
        wid = lax.axis_index("c") * 16 + lax.axis_index("s")
        row = wid // tiles_per_row
        slot = wid % tiles_per_row
        base = row * hw + slot * chunk

        pltpu.sync_copy(t_hbm.at[pl.ds(base, chunk)], vals_v)
        pltpu.sync_copy(cs_hbm.at[row], cs_v)

        fneg = jnp.full((L16,), NEG, jnp.float32)
        fpos = jnp.full((L16,), POS, jnp.float32)
        for g in range(SEG_PAD // L16):
            imax_v[pl.ds(g * L16, L16)] = fneg
            imin_v[pl.ds(g * L16, L16)] = fpos
        im1 = jnp.full((L16,), -1, jnp.int32)
        sbuf_v[pl.ds(0, L16)] = im1
        sbuf_v[pl.ds(L16, L16)] = im1

        def body(g, carry):
            b_acc, l_acc = carry
            v = vals_v[pl.ds(g * L16, L16)]
            valid = v >= 0.001
            l_acc = l_acc + jnp.where(valid, 1.0, 0.0).astype(jnp.float32)
            lo = jnp.zeros((L16,), jnp.int32)
            hi = jnp.full((L16,), p, jnp.int32)
            for _ in range(9):
                mid = (lo + hi) >> 1
                cm = plsc.load_gather(cs_v, [mid + 1])
                go = cm <= v
                lo = jnp.where(go, mid + 1, lo)
                hi = jnp.where(go, hi, mid)
            s = lo
            lower = plsc.load_gather(cs_v, [s])
            upper = plsc.load_gather(cs_v, [s + 1])
            d = jnp.minimum(v - lower, upper - v)
            b_acc = b_acc + jnp.where(valid, d * d, 0.0)
            key = jnp.where(valid, v, 2.0)
            sidx = jnp.where(valid, s, trash)
            ksort, isort = plsc.sort_key_val(key, sidx)
            sbuf_v[pl.ds(1, L16)] = isort
            sprev = sbuf_v[pl.ds(0, L16)]
            snext = sbuf_v[pl.ds(2, L16)]
            m_last = isort != snext
            m_first = isort != sprev
            old_mx = plsc.load_gather(imax_v, [isort])
            plsc.store_scatter(imax_v, [isort], jnp.maximum(old_mx, ksort),
                               mask=m_last)
            old_mn = plsc.load_gather(imin_v, [isort])
            plsc.store_scatter(imin_v, [isort], jnp.minimum(old_mn, ksort),
                               mask=m_first)
            return b_acc, l_acc

        zero16 = jnp.zeros((L16,), jnp.float32)
        b_acc, l_acc = lax.fori_loop(0, groups, body, (zero16, zero16))

        ii = lax.iota(jnp.int32, L16)
        stats_v[...] = jnp.where(ii == 0, jnp.sum(b_acc),
                                 jnp.where(ii == 1, jnp.sum(l_acc), 0.0)
                                 ).astype(jnp.float32)
        pltpu.sync_copy(imax_v, recs_hbm.at[wid, pl.ds(0, SEG_PAD)])
        pltpu.sync_copy(imin_v, recs_hbm.at[wid, pl.ds(SEG_PAD, SEG_PAD)])
        pltpu.sync_copy(stats_v, recs_hbm.at[wid, pl.ds(608, L16)])

    return k(csort_pad, tflat)


def _combine_sc(csort_pad, recs, *, n, p, n_tiles):
    tiles_per_row = n_tiles // n
    cpad = csort_pad.shape[1]
    a_groups = cpad // L16
    c_groups = p // L16

    mesh = plsc.VectorSubcoreMesh(
        core_axis_name="c", subcore_axis_name="s", num_cores=1,
        num_subcores=16,
    )

    @functools.partial(
        pl.kernel,
        out_type=jax.ShapeDtypeStruct((L16,), jnp.float32),
        mesh=mesh,
        compiler_params=pltpu.CompilerParams(
            use_tc_tiling_on_sc=False, needs_layout_passes=False),
        scratch_types=dict(
            row_v=pltpu.VMEM((tiles_per_row, REC), jnp.float32),
            cs_v=pltpu.VMEM((cpad,), jnp.float32),
            cimax_v=pltpu.VMEM((cpad,), jnp.float32),
            cimin_v=pltpu.VMEM((cpad,), jnp.float32),
            pmax_v=pltpu.VMEM((cpad,), jnp.float32),
            smin_v=pltpu.VMEM((cpad,), jnp.float32),
            stats_v=pltpu.VMEM((L16,), jnp.float32),
            res_v=pltpu.VMEM((n, L16), jnp.float32),
            out_v=pltpu.VMEM((L16,), jnp.float32),
            shared=pltpu.VMEM_SHARED((16, L16), jnp.float32),
        ),
    )
    def k(cs_hbm, recs_hbm, out_hbm, row_v, cs_v, cimax_v, cimin_v,
          pmax_v, smin_v, stats_v, res_v, out_v, shared):
        wid = lax.axis_index("s")
        ii = lax.iota(jnp.int32, L16)

        @pl.when(wid < n)
        def _():
            r = wid
            pltpu.sync_copy(recs_hbm.at[pl.ds(r * tiles_per_row,
                                              tiles_per_row)], row_v)
            pltpu.sync_copy(cs_hbm.at[r], cs_v)
            for g in range(a_groups):
                mx = row_v[0, pl.ds(g * L16, L16)]
                mn = row_v[0, pl.ds(SEG_PAD + g * L16, L16)]
                for t in range(1, tiles_per_row):
                    mx = jnp.maximum(mx, row_v[t, pl.ds(g * L16, L16)])
                    mn = jnp.minimum(mn, row_v[t, pl.ds(SEG_PAD + g * L16,
                                                        L16)])
                cimax_v[pl.ds(g * L16, L16)] = mx
                cimin_v[pl.ds(g * L16, L16)] = mn
            st = row_v[0, pl.ds(608, L16)]
            for t in range(1, tiles_per_row):
                st = st + row_v[t, pl.ds(608, L16)]
            b_r = st[0]
            l_r = st[1]
            carry = jnp.float32(NEG)
            for g in range(a_groups):
                x = cimax_v[pl.ds(g * L16, L16)]
                pmax_v[pl.ds(g * L16, L16)] = jnp.maximum(plsc.cummax(x),
                                                          carry)
                carry = jnp.maximum(carry, jnp.max(x))
            carry = jnp.float32(POS)
            for g in reversed(range(a_groups)):
                x = cimin_v[pl.ds(g * L16, L16)]
                suff = -lax.rev(plsc.cummax(lax.rev(-x, (0,))), (0,))
                smin_v[pl.ds(g * L16, L16)] = jnp.minimum(suff, carry)
                carry = jnp.minimum(carry, jnp.min(x))
            sa = jnp.zeros((L16,), jnp.float32)
            sac = jnp.zeros((L16,), jnp.float32)
            for h in range(c_groups):
                cp = cs_v[pl.ds(1 + h * L16, L16)]
                nb = pmax_v[pl.ds(h * L16, L16)]
                na = smin_v[pl.ds(h * L16 + 1, L16)]
                db = cp - nb
                da = na - cp
                a_p = jnp.minimum(db * db, da * da)
                sa = sa + a_p
                sac = sac + jnp.minimum(a_p, cp * cp)
            sa_s = jnp.sum(sa)
            sac_s = jnp.sum(sac)
            c0 = cs_v[pl.ds(0, L16)][1]
            w = jnp.where(ii == 0, sa_s,
                          jnp.where(ii == 1, sac_s,
                                    jnp.where(ii == 2, b_r,
                                              jnp.where(ii == 3, l_r,
                                                        jnp.where(ii == 4, c0,
                                                                  0.0)))))
            stats_v[...] = w.astype(jnp.float32)
            pltpu.sync_copy(stats_v, shared.at[wid])

        plsc.subcore_barrier()

        @pl.when(wid == 0)
        def _():
            pltpu.sync_copy(shared.at[pl.ds(0, n)], res_v)
            rv = [res_v[r, pl.ds(0, L16)] for r in range(n)]
            ls = [rv[r][3] for r in range(n)]
            t_max = ls[0]
            for r in range(1, n):
                t_max = jnp.maximum(t_max, ls[r])
            chx = jnp.float32(0.0)
            chy_num = jnp.float32(0.0)
            for r in range(n):
                sa_s = rv[r][0]
                sac_s = rv[r][1]
                b_r = rv[r][2]
                c0 = rv[r][4]
                chx = chx + jnp.where(t_max > ls[r], sac_s, sa_s) * (1.0 / p)
                chy_num = chy_num + b_r + (t_max - ls[r]) * c0 * c0
            chy_vec = (jnp.broadcast_to(chy_num, (L16,))
                       / jnp.broadcast_to(t_max, (L16,)))
            loss_vec = (0.1 / n) * (chx + chy_vec)
            out_v[...] = jnp.where(ii == 0, loss_vec, 0.0).astype(jnp.float32)
            pltpu.sync_copy(out_v, out_hbm)

    return k(csort_pad, recs)


def kernel(input, target):
    bins = input
    centers = 0.5 * (bins[:, 1:] + bins[:, :-1])
    n, p = centers.shape
    hw = target.shape[1] * target.shape[2]
    csort = jnp.sort(centers, axis=1)
    csort_pad = jnp.concatenate(
        [jnp.full((n, 1), NEG, jnp.float32),
         csort,
         jnp.full((n, 15), POS, jnp.float32)], axis=1)
    tflat = target.reshape(n * hw)
    recs = _pixel_phase_sc(csort_pad, tflat, n=n, p=p, hw=hw, n_tiles=32)
    out = _combine_sc(csort_pad, recs, n=n, p=p, n_tiles=32)
    return out[0]
